# baseline (speedup 1.0000x reference)
import functools

import jax
import jax.numpy as jnp
from jax import lax
from jax.experimental import pallas as pl
from jax.experimental.pallas import tpu as pltpu
from jax.experimental.pallas import tpu_sc as plsc

_NT = (((1,), (1,)), ((), ()))


def _leaky(y):
    return jnp.maximum(y, 0.2 * y)


def _head_body(xT_ref, xF_ref, idx_ref, S_ref, *, TN, NPTS, KNBR, prec):
    b = pl.program_id(0)
    r = pl.program_id(1)
    xT = xT_ref[0]
    xt = xT_ref[0, pl.ds(r * TN, TN), :]
    sc2 = lax.dot_general(xt, xT, _NT, precision=prec)
    xF = xF_ref[0]
    xx = jnp.sum(xF * xF, axis=0, keepdims=True)
    S_ref[...] = 2.0 * sc2 - xx

    colids = lax.broadcasted_iota(jnp.int32, (TN, NPTS), 1)
    lane_k = lax.broadcasted_iota(jnp.int32, (TN, KNBR), 1)
    cid128 = lax.broadcasted_iota(jnp.int32, (TN, 128), 1)
    base = b * NPTS
    NB = NPTS // 128
    NINF = -jnp.inf
    BIG = jnp.int32(NPTS)

    def body(t, acc):
        v1 = jnp.full((TN, 128), NINF, jnp.float32)
        c1 = jnp.full((TN, 128), BIG, jnp.int32)
        v2 = jnp.full((TN, 128), NINF, jnp.float32)
        c2 = jnp.full((TN, 128), BIG, jnp.int32)
        for j in range(NB):
            cur = S_ref[:, j * 128:(j + 1) * 128]
            cid = cid128 + (j * 128)
            gt1 = cur > v1
            gt2 = cur > v2
            v2 = jnp.where(gt1, v1, jnp.where(gt2, cur, v2))
            c2 = jnp.where(gt1, c1, jnp.where(gt2, cid, c2))
            v1 = jnp.where(gt1, cur, v1)
            c1 = jnp.where(gt1, cid, c1)
        m1 = jnp.max(v1, axis=1, keepdims=True)
        am1 = jnp.min(jnp.where(v1 == m1, c1, BIG), axis=1, keepdims=True)
        v1m = jnp.where(c1 == am1, NINF, v1)
        m2 = jnp.maximum(jnp.max(v1m, axis=1, keepdims=True),
                         jnp.max(v2, axis=1, keepdims=True))
        am2 = jnp.minimum(
            jnp.min(jnp.where(v1m == m2, c1, BIG), axis=1, keepdims=True),
            jnp.min(jnp.where(v2 == m2, c2, BIG), axis=1, keepdims=True))
        s = S_ref[...]
        S_ref[...] = jnp.where((colids == am1) | (colids == am2), NINF, s)
        return (acc + jnp.where(lane_k == 2 * t, am1 + base, 0)
                + jnp.where(lane_k == 2 * t + 1, am2 + base, 0))

    idx_ref[0] = lax.fori_loop(0, KNBR // 2, body,
                               jnp.zeros((TN, KNBR), jnp.int32))


def _head(xT, xF, *, prec, TN=512, KNBR=20, interpret=False):
    B, NPTS, d = xT.shape
    return pl.pallas_call(
        functools.partial(_head_body, TN=TN, NPTS=NPTS, KNBR=KNBR, prec=prec),
        grid=(B, NPTS // TN),
        in_specs=[
            pl.BlockSpec((1, NPTS, d), lambda b, r: (b, 0, 0)),
            pl.BlockSpec((1, d, NPTS), lambda b, r: (b, 0, 0)),
        ],
        out_specs=pl.BlockSpec((1, TN, KNBR), lambda b, r: (b, r, 0)),
        out_shape=jax.ShapeDtypeStruct((B, NPTS, KNBR), jnp.int32),
        scratch_shapes=[pltpu.VMEM((TN, NPTS), jnp.float32)],
        interpret=interpret,
    )(xT, xF)


def _gather_rows(table, idx_flat):
    M = idx_flat.shape[0]
    D = table.shape[1]
    NW = 32
    CH = 128
    KB = 8
    per_w = M // NW
    n_ch = per_w // CH
    n_mc = n_ch // KB
    idx2 = idx_flat.reshape(M // CH, CH)
    mesh = plsc.VectorSubcoreMesh(core_axis_name="c", subcore_axis_name="s")

    @functools.partial(
        pl.kernel,
        out_type=jax.ShapeDtypeStruct((M, D), jnp.float32),
        mesh=mesh,
        compiler_params=pltpu.CompilerParams(use_tc_tiling_on_sc=False),
        scratch_types=[
            pltpu.VMEM((n_ch, CH), jnp.int32),
            pltpu.VMEM((KB * CH, D), jnp.float32),
            pltpu.SemaphoreType.DMA,
        ],
    )
    def gk(table_hbm, idx_hbm, out_hbm, idxs_v, rows_v, sem):
        wid = lax.axis_index("s") * 2 + lax.axis_index("c")
        base = wid * per_w
        pltpu.sync_copy(idx_hbm.at[pl.ds(wid * n_ch, n_ch)], idxs_v)

        def body(g, carry):
            for p in range(KB):
                pltpu.async_copy(table_hbm.at[idxs_v.at[g * KB + p]],
                                 rows_v.at[pl.ds(p * CH, CH)], sem)
            for p in range(KB):
                pltpu.make_async_copy(table_hbm.at[idxs_v.at[g * KB + p]],
                                      rows_v.at[pl.ds(p * CH, CH)], sem).wait()
            pltpu.sync_copy(rows_v,
                            out_hbm.at[pl.ds(base + g * KB * CH, KB * CH)])
            return carry

        lax.fori_loop(0, n_mc, body, 0)

    return gk(table, idx2)


def _edge_body(G_ref, xT_ref, W1_ref, s1_ref, t1_ref, W2_ref, s2_ref, t2_ref,
               o_ref, *, KNBR, DT, prec):
    Cx = xT_ref[0][:, :DT]
    acc = None
    for kk in range(KNBR):
        F = jnp.concatenate([G_ref[0, kk][:, :DT] - Cx, Cx], axis=1)
        Y = lax.dot_general(F, W1_ref[...], _NT, precision=prec)
        Y = _leaky(Y * s1_ref[...] + t1_ref[...])
        Y = lax.dot_general(Y, W2_ref[...], _NT, precision=prec)
        Y = _leaky(Y * s2_ref[...] + t2_ref[...])
        acc = Y if acc is None else jnp.maximum(acc, Y)
    o_ref[0] = acc


def _edge(G, xT, W1, s1, t1, W2, s2, t2, *, TM=512, prec, interpret=False):
    B, KNBR, NPTS, d = G.shape
    return pl.pallas_call(
        functools.partial(_edge_body, KNBR=KNBR, DT=W1.shape[1] // 2,
                          prec=prec),
        grid=(B, NPTS // TM),
        in_specs=[
            pl.BlockSpec((1, KNBR, TM, d), lambda b, m: (b, 0, m, 0)),
            pl.BlockSpec((1, TM, d), lambda b, m: (b, m, 0)),
            pl.BlockSpec(W1.shape, lambda b, m: (0, 0)),
            pl.BlockSpec((1, 64), lambda b, m: (0, 0)),
            pl.BlockSpec((1, 64), lambda b, m: (0, 0)),
            pl.BlockSpec((64, 64), lambda b, m: (0, 0)),
            pl.BlockSpec((1, 64), lambda b, m: (0, 0)),
            pl.BlockSpec((1, 64), lambda b, m: (0, 0)),
        ],
        out_specs=pl.BlockSpec((1, TM, 64), lambda b, m: (b, m, 0)),
        out_shape=jax.ShapeDtypeStruct((B, NPTS, 64), jnp.float32),
        interpret=interpret,
    )(G, xT, W1, s1, t1, W2, s2, t2)


def _edge1_body(G_ref, xT_ref, W_ref, s_ref, t_ref, o_ref, *, KNBR, prec):
    Cx = xT_ref[0]
    acc = None
    for kk in range(KNBR):
        F = jnp.concatenate([G_ref[0, kk] - Cx, Cx], axis=1)
        Y = lax.dot_general(F, W_ref[...], _NT, precision=prec)
        Y = Y * s_ref[...] + t_ref[...]
        acc = Y if acc is None else jnp.maximum(acc, Y)
    o_ref[0] = _leaky(acc)


def _edge1(G, xT, W, s, t, *, TM=512, prec, interpret=False):
    B, KNBR, NPTS, d = G.shape
    return pl.pallas_call(
        functools.partial(_edge1_body, KNBR=KNBR, prec=prec),
        grid=(B, NPTS // TM),
        in_specs=[
            pl.BlockSpec((1, KNBR, TM, d), lambda b, m: (b, 0, m, 0)),
            pl.BlockSpec((1, TM, d), lambda b, m: (b, m, 0)),
            pl.BlockSpec(W.shape, lambda b, m: (0, 0)),
            pl.BlockSpec((1, 64), lambda b, m: (0, 0)),
            pl.BlockSpec((1, 64), lambda b, m: (0, 0)),
        ],
        out_specs=pl.BlockSpec((1, TM, 64), lambda b, m: (b, m, 0)),
        out_shape=jax.ShapeDtypeStruct((B, NPTS, 64), jnp.float32),
        interpret=interpret,
    )(G, xT, W, s, t)


def _final_body(cat_ref, W6_ref, s6_ref, t6_ref, out_ref, crit_ref, pool_ref,
                *, NPTS, prec):
    y = lax.dot_general(W6_ref[...], cat_ref[0], _NT, precision=prec)
    y = _leaky(y * s6_ref[...] + t6_ref[...])
    out_ref[0] = y
    m = jnp.max(y, axis=1, keepdims=True)
    cols = lax.broadcasted_iota(jnp.int32, (1024, NPTS), 1)
    crit_ref[0] = jnp.min(jnp.where(y == m, cols, NPTS), axis=1, keepdims=True)
    pool_ref[0] = m


def _final(cat, W6, s6, t6, *, prec, interpret=False):
    B, NPTS, _ = cat.shape
    return pl.pallas_call(
        functools.partial(_final_body, NPTS=NPTS, prec=prec),
        grid=(B,),
        in_specs=[
            pl.BlockSpec((1, NPTS, 192), lambda b: (b, 0, 0)),
            pl.BlockSpec((1024, 192), lambda b: (0, 0)),
            pl.BlockSpec((1024, 1), lambda b: (0, 0)),
            pl.BlockSpec((1024, 1), lambda b: (0, 0)),
        ],
        out_specs=[
            pl.BlockSpec((1, 1024, NPTS), lambda b: (b, 0, 0)),
            pl.BlockSpec((1, 1024, 1), lambda b: (b, 0, 0)),
            pl.BlockSpec((1, 1024, 1), lambda b: (b, 0, 0)),
        ],
        out_shape=[
            jax.ShapeDtypeStruct((B, 1024, NPTS), jnp.float32),
            jax.ShapeDtypeStruct((B, 1024, 1), jnp.int32),
            jax.ShapeDtypeStruct((B, 1024, 1), jnp.float32),
        ],
        interpret=interpret,
    )(cat, W6, s6, t6)


def _pipeline(x, W1, W2, W3, W4, W5, W6, s1, s2, s3, s4, s5, s6,
              t1, t2, t3, t4, t5, t6, gather_fn, interpret=False):
    B, _, NPTS = x.shape
    HI = lax.Precision.HIGHEST
    DE = lax.Precision.DEFAULT
    row = lambda v: v[None, :]

    xT1 = jnp.pad(jnp.transpose(x, (0, 2, 1)), ((0, 0), (0, 0), (0, 13)))
    xF1 = jnp.pad(x, ((0, 0), (0, 5), (0, 0)))
    idx = _head(xT1[:, :, :8], xF1, prec=DE, interpret=interpret)
    idxT = jnp.transpose(idx, (0, 2, 1)).reshape(-1)
    G = gather_fn(xT1.reshape(B * NPTS, 16), idxT).reshape(B, 20, NPTS, 16)
    x1T = _edge(G, xT1, W1, row(s1), row(t1), W2, row(s2), row(t2),
                prec=DE, interpret=interpret)

    x1F = jnp.transpose(x1T, (0, 2, 1))
    idx = _head(x1T, x1F, prec=DE, interpret=interpret)
    idxT = jnp.transpose(idx, (0, 2, 1)).reshape(-1)
    G = gather_fn(x1T.reshape(B * NPTS, 64), idxT).reshape(B, 20, NPTS, 64)
    x2T = _edge(G, x1T, W3, row(s3), row(t3), W4, row(s4), row(t4),
                prec=DE, interpret=interpret)

    x2F = jnp.transpose(x2T, (0, 2, 1))
    idx = _head(x2T, x2F, prec=DE, interpret=interpret)
    idxT = jnp.transpose(idx, (0, 2, 1)).reshape(-1)
    G = gather_fn(x2T.reshape(B * NPTS, 64), idxT).reshape(B, 20, NPTS, 64)
    x3T = _edge1(G, x2T, W5, row(s5), row(t5), prec=DE, interpret=interpret)

    cat = jnp.concatenate([x1T, x2T, x3T], axis=2)
    out, crit, pool = _final(cat, W6, s6[:, None], t6[:, None],
                             prec=DE, interpret=interpret)
    return out, crit.reshape(B, 1024), pool.reshape(B, 1024)


def kernel(x, W1, W2, W3, W4, W5, W6, s1, s2, s3, s4, s5, s6,
           t1, t2, t3, t4, t5, t6):
    return _pipeline(x, W1, W2, W3, W4, W5, W6, s1, s2, s3, s4, s5, s6,
                     t1, t2, t3, t4, t5, t6, _gather_rows)


# speedup vs baseline: 1.1376x; 1.1376x over previous
import functools

import jax
import jax.numpy as jnp
from jax import lax
from jax.experimental import pallas as pl
from jax.experimental.pallas import tpu as pltpu
from jax.experimental.pallas import tpu_sc as plsc

_NT = (((1,), (1,)), ((), ()))


def _leaky(y):
    return jnp.maximum(y, 0.2 * y)


def _head_body(xT_ref, xF_ref, idx_ref, S_ref, *, TN, NPTS, KNBR, prec):
    b = pl.program_id(0)
    r = pl.program_id(1)
    xT = xT_ref[0]
    xt = xT_ref[0, pl.ds(r * TN, TN), :]
    sc2 = lax.dot_general(xt, xT, _NT, precision=prec)
    xF = xF_ref[0]
    xx = jnp.sum(xF * xF, axis=0, keepdims=True)
    S_ref[...] = 2.0 * sc2 - xx

    colids = lax.broadcasted_iota(jnp.int32, (TN, NPTS), 1)
    lane_k = lax.broadcasted_iota(jnp.int32, (TN, KNBR), 1)
    base = b * NPTS

    def body(t, acc):
        s = S_ref[...]
        m = jnp.max(s, axis=1, keepdims=True)
        am = jnp.min(jnp.where(s == m, colids, NPTS), axis=1, keepdims=True)
        S_ref[...] = jnp.where(colids == am, -jnp.inf, s)
        return acc + jnp.where(lane_k == t, am + base, 0)

    idx_ref[0] = lax.fori_loop(0, KNBR, body, jnp.zeros((TN, KNBR), jnp.int32))


def _head(xT, xF, *, prec, TN=512, KNBR=20, interpret=False):
    B, NPTS, d = xT.shape
    return pl.pallas_call(
        functools.partial(_head_body, TN=TN, NPTS=NPTS, KNBR=KNBR, prec=prec),
        grid=(B, NPTS // TN),
        in_specs=[
            pl.BlockSpec((1, NPTS, d), lambda b, r: (b, 0, 0)),
            pl.BlockSpec((1, d, NPTS), lambda b, r: (b, 0, 0)),
        ],
        out_specs=pl.BlockSpec((1, TN, KNBR), lambda b, r: (b, r, 0)),
        out_shape=jax.ShapeDtypeStruct((B, NPTS, KNBR), jnp.int32),
        scratch_shapes=[pltpu.VMEM((TN, NPTS), jnp.float32)],
        interpret=interpret,
    )(xT, xF)


def _gather_rows(table, idx_flat):
    M = idx_flat.shape[0]
    D = table.shape[1]
    NW = 32
    CH = 128
    KB = 8
    per_w = M // NW
    n_ch = per_w // CH
    n_mc = n_ch // KB
    idx2 = idx_flat.reshape(M // CH, CH)
    mesh = plsc.VectorSubcoreMesh(core_axis_name="c", subcore_axis_name="s")

    @functools.partial(
        pl.kernel,
        out_type=jax.ShapeDtypeStruct((M, D), jnp.float32),
        mesh=mesh,
        compiler_params=pltpu.CompilerParams(use_tc_tiling_on_sc=False),
        scratch_types=[
            pltpu.VMEM((n_ch, CH), jnp.int32),
            pltpu.VMEM((KB * CH, D), jnp.float32),
            pltpu.SemaphoreType.DMA,
        ],
    )
    def gk(table_hbm, idx_hbm, out_hbm, idxs_v, rows_v, sem):
        wid = lax.axis_index("s") * 2 + lax.axis_index("c")
        base = wid * per_w
        pltpu.sync_copy(idx_hbm.at[pl.ds(wid * n_ch, n_ch)], idxs_v)

        def body(g, carry):
            for p in range(KB):
                pltpu.async_copy(table_hbm.at[idxs_v.at[g * KB + p]],
                                 rows_v.at[pl.ds(p * CH, CH)], sem)
            for p in range(KB):
                pltpu.make_async_copy(table_hbm.at[idxs_v.at[g * KB + p]],
                                      rows_v.at[pl.ds(p * CH, CH)], sem).wait()
            pltpu.sync_copy(rows_v,
                            out_hbm.at[pl.ds(base + g * KB * CH, KB * CH)])
            return carry

        lax.fori_loop(0, n_mc, body, 0)

    return gk(table, idx2)


def _edge_body(G_ref, xT_ref, W1_ref, s1_ref, t1_ref, W2_ref, s2_ref, t2_ref,
               o_ref, *, KNBR, DT, prec):
    Cx = xT_ref[0][:, :DT]
    acc = None
    for kk in range(KNBR):
        F = jnp.concatenate([G_ref[0, kk][:, :DT] - Cx, Cx], axis=1)
        Y = lax.dot_general(F, W1_ref[...], _NT, precision=prec)
        Y = _leaky(Y * s1_ref[...] + t1_ref[...])
        Y = lax.dot_general(Y, W2_ref[...], _NT, precision=prec)
        Y = _leaky(Y * s2_ref[...] + t2_ref[...])
        acc = Y if acc is None else jnp.maximum(acc, Y)
    o_ref[0] = acc


def _edge(G, xT, W1, s1, t1, W2, s2, t2, *, TM=1024, prec, interpret=False):
    B, KNBR, NPTS, d = G.shape
    return pl.pallas_call(
        functools.partial(_edge_body, KNBR=KNBR, DT=W1.shape[1] // 2,
                          prec=prec),
        grid=(B, NPTS // TM),
        in_specs=[
            pl.BlockSpec((1, KNBR, TM, d), lambda b, m: (b, 0, m, 0)),
            pl.BlockSpec((1, TM, d), lambda b, m: (b, m, 0)),
            pl.BlockSpec(W1.shape, lambda b, m: (0, 0)),
            pl.BlockSpec((1, 64), lambda b, m: (0, 0)),
            pl.BlockSpec((1, 64), lambda b, m: (0, 0)),
            pl.BlockSpec((64, 64), lambda b, m: (0, 0)),
            pl.BlockSpec((1, 64), lambda b, m: (0, 0)),
            pl.BlockSpec((1, 64), lambda b, m: (0, 0)),
        ],
        out_specs=pl.BlockSpec((1, TM, 64), lambda b, m: (b, m, 0)),
        out_shape=jax.ShapeDtypeStruct((B, NPTS, 64), jnp.float32),
        interpret=interpret,
    )(G, xT, W1, s1, t1, W2, s2, t2)


def _edge1_body(G_ref, xT_ref, W_ref, s_ref, t_ref, o_ref, *, KNBR, prec):
    Cx = xT_ref[0]
    acc = None
    for kk in range(KNBR):
        F = jnp.concatenate([G_ref[0, kk] - Cx, Cx], axis=1)
        Y = lax.dot_general(F, W_ref[...], _NT, precision=prec)
        Y = Y * s_ref[...] + t_ref[...]
        acc = Y if acc is None else jnp.maximum(acc, Y)
    o_ref[0] = _leaky(acc)


def _edge1(G, xT, W, s, t, *, TM=1024, prec, interpret=False):
    B, KNBR, NPTS, d = G.shape
    return pl.pallas_call(
        functools.partial(_edge1_body, KNBR=KNBR, prec=prec),
        grid=(B, NPTS // TM),
        in_specs=[
            pl.BlockSpec((1, KNBR, TM, d), lambda b, m: (b, 0, m, 0)),
            pl.BlockSpec((1, TM, d), lambda b, m: (b, m, 0)),
            pl.BlockSpec(W.shape, lambda b, m: (0, 0)),
            pl.BlockSpec((1, 64), lambda b, m: (0, 0)),
            pl.BlockSpec((1, 64), lambda b, m: (0, 0)),
        ],
        out_specs=pl.BlockSpec((1, TM, 64), lambda b, m: (b, m, 0)),
        out_shape=jax.ShapeDtypeStruct((B, NPTS, 64), jnp.float32),
        interpret=interpret,
    )(G, xT, W, s, t)


def _final_body(cat_ref, W6_ref, s6_ref, t6_ref, out_ref, crit_ref, pool_ref,
                *, NPTS, prec):
    y = lax.dot_general(W6_ref[...], cat_ref[0], _NT, precision=prec)
    y = _leaky(y * s6_ref[...] + t6_ref[...])
    out_ref[0] = y
    m = jnp.max(y, axis=1, keepdims=True)
    cols = lax.broadcasted_iota(jnp.int32, (1024, NPTS), 1)
    crit_ref[0] = jnp.min(jnp.where(y == m, cols, NPTS), axis=1, keepdims=True)
    pool_ref[0] = m


def _final(cat, W6, s6, t6, *, prec, interpret=False):
    B, NPTS, _ = cat.shape
    return pl.pallas_call(
        functools.partial(_final_body, NPTS=NPTS, prec=prec),
        grid=(B,),
        in_specs=[
            pl.BlockSpec((1, NPTS, 192), lambda b: (b, 0, 0)),
            pl.BlockSpec((1024, 192), lambda b: (0, 0)),
            pl.BlockSpec((1024, 1), lambda b: (0, 0)),
            pl.BlockSpec((1024, 1), lambda b: (0, 0)),
        ],
        out_specs=[
            pl.BlockSpec((1, 1024, NPTS), lambda b: (b, 0, 0)),
            pl.BlockSpec((1, 1024, 1), lambda b: (b, 0, 0)),
            pl.BlockSpec((1, 1024, 1), lambda b: (b, 0, 0)),
        ],
        out_shape=[
            jax.ShapeDtypeStruct((B, 1024, NPTS), jnp.float32),
            jax.ShapeDtypeStruct((B, 1024, 1), jnp.int32),
            jax.ShapeDtypeStruct((B, 1024, 1), jnp.float32),
        ],
        interpret=interpret,
    )(cat, W6, s6, t6)


def _pipeline(x, W1, W2, W3, W4, W5, W6, s1, s2, s3, s4, s5, s6,
              t1, t2, t3, t4, t5, t6, gather_fn, interpret=False):
    B, _, NPTS = x.shape
    HI = lax.Precision.HIGHEST
    DE = lax.Precision.DEFAULT
    row = lambda v: v[None, :]

    xT1 = jnp.pad(jnp.transpose(x, (0, 2, 1)), ((0, 0), (0, 0), (0, 13)))
    xF1 = jnp.pad(x, ((0, 0), (0, 5), (0, 0)))
    idx = _head(xT1[:, :, :8], xF1, prec=DE, interpret=interpret)
    idxT = jnp.transpose(idx, (0, 2, 1)).reshape(-1)
    G = gather_fn(xT1.reshape(B * NPTS, 16), idxT).reshape(B, 20, NPTS, 16)
    x1T = _edge(G, xT1, W1, row(s1), row(t1), W2, row(s2), row(t2),
                prec=DE, interpret=interpret)

    x1F = jnp.transpose(x1T, (0, 2, 1))
    idx = _head(x1T, x1F, prec=DE, interpret=interpret)
    idxT = jnp.transpose(idx, (0, 2, 1)).reshape(-1)
    G = gather_fn(x1T.reshape(B * NPTS, 64), idxT).reshape(B, 20, NPTS, 64)
    x2T = _edge(G, x1T, W3, row(s3), row(t3), W4, row(s4), row(t4),
                prec=DE, interpret=interpret)

    x2F = jnp.transpose(x2T, (0, 2, 1))
    idx = _head(x2T, x2F, prec=DE, interpret=interpret)
    idxT = jnp.transpose(idx, (0, 2, 1)).reshape(-1)
    G = gather_fn(x2T.reshape(B * NPTS, 64), idxT).reshape(B, 20, NPTS, 64)
    x3T = _edge1(G, x2T, W5, row(s5), row(t5), prec=DE, interpret=interpret)

    cat = jnp.concatenate([x1T, x2T, x3T], axis=2)
    out, crit, pool = _final(cat, W6, s6[:, None], t6[:, None],
                             prec=DE, interpret=interpret)
    return out, crit.reshape(B, 1024), pool.reshape(B, 1024)


def kernel(x, W1, W2, W3, W4, W5, W6, s1, s2, s3, s4, s5, s6,
           t1, t2, t3, t4, t5, t6):
    return _pipeline(x, W1, W2, W3, W4, W5, W6, s1, s2, s3, s4, s5, s6,
                     t1, t2, t3, t4, t5, t6, _gather_rows)
